# baseline (speedup 1.0000x reference)
import functools

import jax
import jax.numpy as jnp
from jax import lax
from jax.experimental import pallas as pl
from jax.experimental.pallas import tpu as pltpu
from jax.experimental.pallas import tpu_sc as plsc

N_NODES = 10000
N_PAD = 10240
D = 128
N_EDGES = 320000
NC, NS = 2, 16
NW = NC * NS
EPW = N_EDGES // NW
STEP = 64
NSTEP = 160
NH = 4
HS = NSTEP // NH
NB = 4
EPW_PAD = NSTEP * STEP
DSTEP = 128
DNSTEP = EPW_PAD // DSTEP
ROWS_PER_TILE = N_PAD // NS

_mesh = plsc.VectorSubcoreMesh(core_axis_name="c", subcore_axis_name="s")


def _deg_body(dst_hbm, out_hbm, dst_v, ones_v, zbuf_v, deg_sh, sem):
    c = lax.axis_index("c")
    s = lax.axis_index("s")
    wid = c * NS + s
    pltpu.sync_copy(dst_hbm.at[wid], dst_v)

    def fill(i, _):
        ones_v[i, :] = jnp.full((16,), 1.0, jnp.float32)
        zbuf_v[i, :] = jnp.zeros((16,), jnp.float32)
        return 0
    lax.fori_loop(0, DSTEP, fill, 0)
    for b in range(ROWS_PER_TILE // DSTEP):
        pltpu.sync_copy(zbuf_v, deg_sh.at[pl.ds(s * ROWS_PER_TILE + b * DSTEP, DSTEP)])
    plsc.subcore_barrier()

    def step(j, _):
        pltpu.sync_copy(ones_v, deg_sh.at[dst_v.at[j]], add=True)
        return 0
    lax.fori_loop(0, DNSTEP, step, 0)
    plsc.subcore_barrier()
    pltpu.sync_copy(deg_sh.at[pl.ds(s * ROWS_PER_TILE, ROWS_PER_TILE)],
                    out_hbm.at[c, pl.ds(s * ROWS_PER_TILE, ROWS_PER_TILE)])


_deg_kernel = functools.partial(
    pl.kernel,
    out_type=jax.ShapeDtypeStruct((NC, N_PAD, 16), jnp.float32),
    mesh=_mesh,
    scratch_types=[
        pltpu.VMEM((DNSTEP, DSTEP), jnp.int32),
        pltpu.VMEM((DSTEP, 16), jnp.float32),
        pltpu.VMEM((DSTEP, 16), jnp.float32),
        pltpu.VMEM_SHARED((N_PAD, 16), jnp.float32),
        pltpu.SemaphoreType.DMA,
    ],
)(_deg_body)


def _acc_body(table_hbm, src_hbm, dst_hbm, out_hbm,
              src_v, dst_v, r0, r1, r2, r3, gsem_s, ssem_s, acc_sh):
    rows = (r0, r1, r2, r3)
    gsem = (gsem_s,) * NB
    ssem = (ssem_s,) * NB
    c = lax.axis_index("c")
    s = lax.axis_index("s")
    wid = c * NS + s

    def zrow(i, _):
        for k in range(D // 16):
            rows[0][i, pl.ds(16 * k, 16)] = jnp.zeros((16,), jnp.float32)
        return 0
    lax.fori_loop(0, STEP, zrow, 0)
    for b in range(ROWS_PER_TILE // STEP):
        pltpu.sync_copy(rows[0],
                        acc_sh.at[pl.ds(s * ROWS_PER_TILE + b * STEP, STEP)])
    plsc.subcore_barrier()

    def gat(j, k):
        pltpu.async_copy(table_hbm.at[src_v.at[j]], rows[k], gsem[k])

    def gwait(j, k):
        pltpu.make_async_copy(table_hbm.at[src_v.at[j]], rows[k], gsem[k]).wait()

    def sca(j, k):
        pltpu.async_copy(rows[k], acc_sh.at[dst_v.at[j]], ssem[k], add=True)

    def swait(j, k):
        pltpu.make_async_copy(rows[k], acc_sh.at[dst_v.at[j]], ssem[k]).wait()

    for h in range(NH):
        pltpu.sync_copy(src_hbm.at[wid, h], src_v)
        pltpu.sync_copy(dst_hbm.at[wid, h], dst_v)
        for k in range(NB):
            gat(k, k)
        for j in (0, 1):
            gwait(j, j % NB)
            sca(j, j % NB)

        def grp(g, _):
            for k in range(NB):
                j = 4 * g + k + 2
                kb = (k + 2) % NB
                gwait(j, kb)
                sca(j, kb)
                swait(j - 2, k)
                gat(j + 2, k)
            return 0
        lax.fori_loop(0, (HS - 4) // NB, grp, 0)
        for j in (HS - 2, HS - 1):
            gwait(j, j % NB)
            sca(j, j % NB)
            swait(j - 2, (j - 2) % NB)
        for j in (HS - 2, HS - 1):
            swait(j, j % NB)

    plsc.subcore_barrier()
    pltpu.sync_copy(acc_sh.at[pl.ds(s * ROWS_PER_TILE, ROWS_PER_TILE)],
                    out_hbm.at[c, pl.ds(s * ROWS_PER_TILE, ROWS_PER_TILE)])


_acc_kernel = functools.partial(
    pl.kernel,
    out_type=jax.ShapeDtypeStruct((NC, N_PAD, D), jnp.float32),
    mesh=_mesh,
    scratch_types=[
        pltpu.VMEM((HS, STEP), jnp.int32),
        pltpu.VMEM((HS, STEP), jnp.int32),
        pltpu.VMEM((STEP, D), jnp.float32),
        pltpu.VMEM((STEP, D), jnp.float32),
        pltpu.VMEM((STEP, D), jnp.float32),
        pltpu.VMEM((STEP, D), jnp.float32),
        pltpu.SemaphoreType.DMA,
        pltpu.SemaphoreType.DMA,
        pltpu.VMEM_SHARED((N_PAD, D), jnp.float32),
    ],
)(_acc_body)


_BLK = 1024
_GRID = N_PAD // _BLK


def _tc1_body(p_ref, x_ref, w_ref, dinv_ref, h1s_ref):
    p = p_ref[...]
    deg = p[0, :, 0:1] + p[1, :, 0:1] + 1.0
    dinv = lax.rsqrt(deg)
    h = jnp.dot(x_ref[...], w_ref[...], preferred_element_type=jnp.float32)
    dinv_ref[...] = dinv
    h1s_ref[...] = dinv * h


def _tc1(deg_part, x_pad, W1):
    return pl.pallas_call(
        _tc1_body,
        grid=(_GRID,),
        in_specs=[
            pl.BlockSpec((NC, _BLK, 16), lambda i: (0, i, 0)),
            pl.BlockSpec((_BLK, D), lambda i: (i, 0)),
            pl.BlockSpec((D, D), lambda i: (0, 0)),
        ],
        out_specs=[
            pl.BlockSpec((_BLK, 1), lambda i: (i, 0)),
            pl.BlockSpec((_BLK, D), lambda i: (i, 0)),
        ],
        out_shape=[
            jax.ShapeDtypeStruct((N_PAD, 1), jnp.float32),
            jax.ShapeDtypeStruct((N_PAD, D), jnp.float32),
        ],
    )(deg_part, x_pad, W1)


def _tc2_body(p_ref, h1s_ref, dinv_ref, b_ref, w_ref, h2s_ref):
    acc = p_ref[0] + p_ref[1]
    dinv = dinv_ref[...]
    out1 = dinv * (acc + h1s_ref[...]) + b_ref[...]
    h2s_ref[...] = dinv * jnp.dot(out1, w_ref[...], preferred_element_type=jnp.float32)


def _tc2(p, h1s, dinv, b1, W2):
    return pl.pallas_call(
        _tc2_body,
        grid=(_GRID,),
        in_specs=[
            pl.BlockSpec((NC, _BLK, D), lambda i: (0, i, 0)),
            pl.BlockSpec((_BLK, D), lambda i: (i, 0)),
            pl.BlockSpec((_BLK, 1), lambda i: (i, 0)),
            pl.BlockSpec((D,), lambda i: (0,)),
            pl.BlockSpec((D, D), lambda i: (0, 0)),
        ],
        out_specs=pl.BlockSpec((_BLK, D), lambda i: (i, 0)),
        out_shape=jax.ShapeDtypeStruct((N_PAD, D), jnp.float32),
    )(p, h1s, dinv, b1, W2)


def _tc3_body(q_ref, h2s_ref, dinv_ref, b_ref, out_ref):
    acc = q_ref[0] + q_ref[1]
    out_ref[...] = dinv_ref[...] * (acc + h2s_ref[...]) + b_ref[...]


def _tc3(q, h2s, dinv, b2):
    return pl.pallas_call(
        _tc3_body,
        grid=(_GRID,),
        in_specs=[
            pl.BlockSpec((NC, _BLK, D), lambda i: (0, i, 0)),
            pl.BlockSpec((_BLK, D), lambda i: (i, 0)),
            pl.BlockSpec((_BLK, 1), lambda i: (i, 0)),
            pl.BlockSpec((D,), lambda i: (0,)),
        ],
        out_specs=pl.BlockSpec((_BLK, D), lambda i: (i, 0)),
        out_shape=jax.ShapeDtypeStruct((N_PAD, D), jnp.float32),
    )(q, h2s, dinv, b2)


def kernel(x, edge_index, W1, b1, W2, b2):
    src = edge_index[0].astype(jnp.int32)
    dst = edge_index[1].astype(jnp.int32)
    npad = NW * EPW_PAD - N_EDGES
    pad_ids = jnp.arange(npad, dtype=jnp.int32)
    src_p = jnp.concatenate([src, pad_ids % N_NODES]).reshape(NW, NH, HS, STEP)
    dst_p = jnp.concatenate([dst, N_NODES + pad_ids % (N_PAD - N_NODES)]
                            ).reshape(NW, NH, HS, STEP)
    x_pad = jnp.pad(x, ((0, N_PAD - N_NODES), (0, 0)))

    deg_part = _deg_kernel(dst_p.reshape(NW, DNSTEP, DSTEP))
    dinv, h1s = _tc1(deg_part, x_pad, W1)
    p1 = _acc_kernel(h1s, src_p, dst_p)
    h2s = _tc2(p1, h1s, dinv, b1, W2)
    p2 = _acc_kernel(h2s, src_p, dst_p)
    out = _tc3(p2, h2s, dinv, b2)
    return out[:N_NODES]


# speedup vs baseline: 1.0432x; 1.0432x over previous
import functools

import jax
import jax.numpy as jnp
from jax import lax
from jax.experimental import pallas as pl
from jax.experimental.pallas import tpu as pltpu
from jax.experimental.pallas import tpu_sc as plsc

N_NODES = 10000
N_PAD = 10240
D = 128
N_EDGES = 320000
NC, NS = 2, 16
NW = NC * NS
EPW = N_EDGES // NW
STEP = 128
NSTEP = 80
NH = 2
HS = NSTEP // NH
EPW_PAD = NSTEP * STEP
ROWS_PER_TILE = N_PAD // NS

_mesh = plsc.VectorSubcoreMesh(core_axis_name="c", subcore_axis_name="s")


def _deg_body(dst_hbm, out_hbm, dst_v, ones_v, zbuf_v, deg_sh, sem):
    c = lax.axis_index("c")
    s = lax.axis_index("s")
    wid = c * NS + s
    pltpu.sync_copy(dst_hbm.at[wid], dst_v)

    def fill(i, _):
        ones_v[i, :] = jnp.full((16,), 1.0, jnp.float32)
        zbuf_v[i, :] = jnp.zeros((16,), jnp.float32)
        return 0
    lax.fori_loop(0, STEP, fill, 0)
    for b in range(ROWS_PER_TILE // STEP):
        pltpu.sync_copy(zbuf_v, deg_sh.at[pl.ds(s * ROWS_PER_TILE + b * STEP, STEP)])
    plsc.subcore_barrier()

    pltpu.async_copy(ones_v, deg_sh.at[dst_v.at[0]], sem, add=True)

    def step(j, _):
        pltpu.async_copy(ones_v, deg_sh.at[dst_v.at[j + 1]], sem, add=True)
        pltpu.make_async_copy(ones_v, deg_sh.at[dst_v.at[j]], sem).wait()
        return 0
    lax.fori_loop(0, NSTEP - 1, step, 0)
    pltpu.make_async_copy(ones_v, deg_sh.at[dst_v.at[NSTEP - 1]], sem).wait()
    plsc.subcore_barrier()
    pltpu.sync_copy(deg_sh.at[pl.ds(s * ROWS_PER_TILE, ROWS_PER_TILE)],
                    out_hbm.at[c, pl.ds(s * ROWS_PER_TILE, ROWS_PER_TILE)])


_deg_kernel = functools.partial(
    pl.kernel,
    out_type=jax.ShapeDtypeStruct((NC, N_PAD, 16), jnp.float32),
    mesh=_mesh,
    scratch_types=[
        pltpu.VMEM((NSTEP, STEP), jnp.int32),
        pltpu.VMEM((STEP, 16), jnp.float32),
        pltpu.VMEM((STEP, 16), jnp.float32),
        pltpu.VMEM_SHARED((N_PAD, 16), jnp.float32),
        pltpu.SemaphoreType.DMA,
    ],
)(_deg_body)


def _acc_body(table_hbm, src_hbm, dst_hbm, out_hbm,
              src_v, dst_v, rows0_v, rows1_v, acc_sh, sem0, sem1):
    c = lax.axis_index("c")
    s = lax.axis_index("s")
    wid = c * NS + s

    def zrow(i, _):
        for k in range(D // 16):
            rows0_v[i, pl.ds(16 * k, 16)] = jnp.zeros((16,), jnp.float32)
        return 0
    lax.fori_loop(0, STEP, zrow, 0)
    for b in range(ROWS_PER_TILE // STEP):
        pltpu.sync_copy(rows0_v,
                        acc_sh.at[pl.ds(s * ROWS_PER_TILE + b * STEP, STEP)])
    plsc.subcore_barrier()

    def gat(j, buf, sm):
        return pltpu.async_copy(table_hbm.at[src_v.at[j]], buf, sm)

    def sca(j, buf):
        pltpu.sync_copy(buf, acc_sh.at[dst_v.at[j]], add=True)

    for h in range(NH):
        pltpu.sync_copy(src_hbm.at[wid, h], src_v)
        pltpu.sync_copy(dst_hbm.at[wid, h], dst_v)
        gat(0, rows0_v, sem0)

        def pair(p, _):
            gat(2 * p + 1, rows1_v, sem1)
            pltpu.make_async_copy(table_hbm.at[src_v.at[2 * p]],
                                  rows0_v, sem0).wait()
            sca(2 * p, rows0_v)
            gat(2 * p + 2, rows0_v, sem0)
            pltpu.make_async_copy(table_hbm.at[src_v.at[2 * p + 1]],
                                  rows1_v, sem1).wait()
            sca(2 * p + 1, rows1_v)
            return 0
        lax.fori_loop(0, HS // 2 - 1, pair, 0)
        gat(HS - 1, rows1_v, sem1)
        pltpu.make_async_copy(table_hbm.at[src_v.at[HS - 2]],
                              rows0_v, sem0).wait()
        sca(HS - 2, rows0_v)
        pltpu.make_async_copy(table_hbm.at[src_v.at[HS - 1]],
                              rows1_v, sem1).wait()
        sca(HS - 1, rows1_v)

    plsc.subcore_barrier()
    pltpu.sync_copy(acc_sh.at[pl.ds(s * ROWS_PER_TILE, ROWS_PER_TILE)],
                    out_hbm.at[c, pl.ds(s * ROWS_PER_TILE, ROWS_PER_TILE)])


_acc_kernel = functools.partial(
    pl.kernel,
    out_type=jax.ShapeDtypeStruct((NC, N_PAD, D), jnp.float32),
    mesh=_mesh,
    scratch_types=[
        pltpu.VMEM((HS, STEP), jnp.int32),
        pltpu.VMEM((HS, STEP), jnp.int32),
        pltpu.VMEM((STEP, D), jnp.float32),
        pltpu.VMEM((STEP, D), jnp.float32),
        pltpu.VMEM_SHARED((N_PAD, D), jnp.float32),
        pltpu.SemaphoreType.DMA,
        pltpu.SemaphoreType.DMA,
    ],
)(_acc_body)


_BLK = 1024
_GRID = N_PAD // _BLK


def _tc1_body(p_ref, x_ref, w_ref, dinv_ref, h1s_ref):
    p = p_ref[...]
    deg = p[0, :, 0:1] + p[1, :, 0:1] + 1.0
    dinv = lax.rsqrt(deg)
    h = jnp.dot(x_ref[...], w_ref[...], preferred_element_type=jnp.float32)
    dinv_ref[...] = dinv
    h1s_ref[...] = dinv * h


def _tc1(deg_part, x_pad, W1):
    return pl.pallas_call(
        _tc1_body,
        grid=(_GRID,),
        in_specs=[
            pl.BlockSpec((NC, _BLK, 16), lambda i: (0, i, 0)),
            pl.BlockSpec((_BLK, D), lambda i: (i, 0)),
            pl.BlockSpec((D, D), lambda i: (0, 0)),
        ],
        out_specs=[
            pl.BlockSpec((_BLK, 1), lambda i: (i, 0)),
            pl.BlockSpec((_BLK, D), lambda i: (i, 0)),
        ],
        out_shape=[
            jax.ShapeDtypeStruct((N_PAD, 1), jnp.float32),
            jax.ShapeDtypeStruct((N_PAD, D), jnp.float32),
        ],
    )(deg_part, x_pad, W1)


def _tc2_body(p_ref, h1s_ref, dinv_ref, b_ref, w_ref, h2s_ref):
    acc = p_ref[0] + p_ref[1]
    dinv = dinv_ref[...]
    out1 = dinv * (acc + h1s_ref[...]) + b_ref[...]
    h2s_ref[...] = dinv * jnp.dot(out1, w_ref[...], preferred_element_type=jnp.float32)


def _tc2(p, h1s, dinv, b1, W2):
    return pl.pallas_call(
        _tc2_body,
        grid=(_GRID,),
        in_specs=[
            pl.BlockSpec((NC, _BLK, D), lambda i: (0, i, 0)),
            pl.BlockSpec((_BLK, D), lambda i: (i, 0)),
            pl.BlockSpec((_BLK, 1), lambda i: (i, 0)),
            pl.BlockSpec((D,), lambda i: (0,)),
            pl.BlockSpec((D, D), lambda i: (0, 0)),
        ],
        out_specs=pl.BlockSpec((_BLK, D), lambda i: (i, 0)),
        out_shape=jax.ShapeDtypeStruct((N_PAD, D), jnp.float32),
    )(p, h1s, dinv, b1, W2)


def _tc3_body(q_ref, h2s_ref, dinv_ref, b_ref, out_ref):
    acc = q_ref[0] + q_ref[1]
    out_ref[...] = dinv_ref[...] * (acc + h2s_ref[...]) + b_ref[...]


def _tc3(q, h2s, dinv, b2):
    return pl.pallas_call(
        _tc3_body,
        grid=(_GRID,),
        in_specs=[
            pl.BlockSpec((NC, _BLK, D), lambda i: (0, i, 0)),
            pl.BlockSpec((_BLK, D), lambda i: (i, 0)),
            pl.BlockSpec((_BLK, 1), lambda i: (i, 0)),
            pl.BlockSpec((D,), lambda i: (0,)),
        ],
        out_specs=pl.BlockSpec((_BLK, D), lambda i: (i, 0)),
        out_shape=jax.ShapeDtypeStruct((N_PAD, D), jnp.float32),
    )(q, h2s, dinv, b2)


def kernel(x, edge_index, W1, b1, W2, b2):
    src = edge_index[0].astype(jnp.int32)
    dst = edge_index[1].astype(jnp.int32)
    npad = NW * EPW_PAD - N_EDGES
    pad_ids = jnp.arange(npad, dtype=jnp.int32)
    src_p = jnp.concatenate([src, pad_ids % N_NODES]).reshape(NW, NH, HS, STEP)
    dst_p = jnp.concatenate([dst, N_NODES + pad_ids % (N_PAD - N_NODES)]
                            ).reshape(NW, NH, HS, STEP)
    x_pad = jnp.pad(x, ((0, N_PAD - N_NODES), (0, 0)))

    deg_part = _deg_kernel(dst_p.reshape(NW, NSTEP, STEP))
    dinv, h1s = _tc1(deg_part, x_pad, W1)
    p1 = _acc_kernel(h1s, src_p, dst_p)
    h2s = _tc2(p1, h1s, dinv, b1, W2)
    p2 = _acc_kernel(h2s, src_p, dst_p)
    out = _tc3(p2, h2s, dinv, b2)
    return out[:N_NODES]


# speedup vs baseline: 1.0507x; 1.0072x over previous
import functools

import jax
import jax.numpy as jnp
from jax import lax
from jax.experimental import pallas as pl
from jax.experimental.pallas import tpu as pltpu
from jax.experimental.pallas import tpu_sc as plsc

N_NODES = 10000
N_PAD = 10240
D = 128
N_EDGES = 320000
NC, NS = 2, 16
NW = NC * NS
EPW = N_EDGES // NW
STEP = 128
NSTEP = 80
NH = 2
HS = NSTEP // NH
EPW_PAD = NSTEP * STEP
ROWS_PER_TILE = N_PAD // NS

_mesh = plsc.VectorSubcoreMesh(core_axis_name="c", subcore_axis_name="s")


def _deg_body(dst_hbm, out_hbm, dst_v, ones_v, zbuf_v, deg_sh, sem):
    c = lax.axis_index("c")
    s = lax.axis_index("s")
    wid = c * NS + s
    pltpu.sync_copy(dst_hbm.at[wid], dst_v)

    def fill(i, _):
        ones_v[i, :] = jnp.full((16,), 1.0, jnp.float32)
        zbuf_v[i, :] = jnp.zeros((16,), jnp.float32)
        return 0
    lax.fori_loop(0, STEP, fill, 0)
    for b in range(ROWS_PER_TILE // STEP):
        pltpu.sync_copy(zbuf_v, deg_sh.at[pl.ds(s * ROWS_PER_TILE + b * STEP, STEP)])
    plsc.subcore_barrier()

    pltpu.async_copy(ones_v, deg_sh.at[dst_v.at[0]], sem, add=True)

    def step(j, _):
        pltpu.async_copy(ones_v, deg_sh.at[dst_v.at[j + 1]], sem, add=True)
        pltpu.make_async_copy(ones_v, deg_sh.at[dst_v.at[j]], sem).wait()
        return 0
    lax.fori_loop(0, NSTEP - 1, step, 0)
    pltpu.make_async_copy(ones_v, deg_sh.at[dst_v.at[NSTEP - 1]], sem).wait()
    plsc.subcore_barrier()
    pltpu.sync_copy(deg_sh.at[pl.ds(s * ROWS_PER_TILE, ROWS_PER_TILE)],
                    out_hbm.at[c, pl.ds(s * ROWS_PER_TILE, ROWS_PER_TILE)])


_deg_kernel = functools.partial(
    pl.kernel,
    out_type=jax.ShapeDtypeStruct((NC, N_PAD, 16), jnp.float32),
    mesh=_mesh,
    scratch_types=[
        pltpu.VMEM((NSTEP, STEP), jnp.int32),
        pltpu.VMEM((STEP, 16), jnp.float32),
        pltpu.VMEM((STEP, 16), jnp.float32),
        pltpu.VMEM_SHARED((N_PAD, 16), jnp.float32),
        pltpu.SemaphoreType.DMA,
    ],
)(_deg_body)


def _acc_body(table_hbm, src_hbm, dst_hbm, out_hbm,
              src_v, dst_v, rows0_v, rows1_v, acc_sh, sem0, sem1):
    c = lax.axis_index("c")
    s = lax.axis_index("s")
    wid = c * NS + s

    def zrow(i, _):
        for k in range(D // 16):
            rows0_v[i, pl.ds(16 * k, 16)] = jnp.zeros((16,), jnp.float32)
        return 0
    lax.fori_loop(0, STEP, zrow, 0)
    for b in range(ROWS_PER_TILE // STEP):
        pltpu.sync_copy(rows0_v,
                        acc_sh.at[pl.ds(s * ROWS_PER_TILE + b * STEP, STEP)])
    plsc.subcore_barrier()

    def gat(j, buf, sm):
        return pltpu.async_copy(table_hbm.at[src_v.at[j]], buf, sm)

    def sca(j, buf):
        pltpu.sync_copy(buf, acc_sh.at[dst_v.at[j]], add=True)

    for h in range(NH):
        pltpu.sync_copy(src_hbm.at[wid, h], src_v)
        pltpu.sync_copy(dst_hbm.at[wid, h], dst_v)
        gat(0, rows0_v, sem0)

        def pair(p, _):
            gat(2 * p + 1, rows1_v, sem1)
            pltpu.make_async_copy(table_hbm.at[src_v.at[2 * p]],
                                  rows0_v, sem0).wait()
            sca(2 * p, rows0_v)
            gat(2 * p + 2, rows0_v, sem0)
            pltpu.make_async_copy(table_hbm.at[src_v.at[2 * p + 1]],
                                  rows1_v, sem1).wait()
            sca(2 * p + 1, rows1_v)
            return 0
        lax.fori_loop(0, HS // 2 - 1, pair, 0)
        gat(HS - 1, rows1_v, sem1)
        pltpu.make_async_copy(table_hbm.at[src_v.at[HS - 2]],
                              rows0_v, sem0).wait()
        sca(HS - 2, rows0_v)
        pltpu.make_async_copy(table_hbm.at[src_v.at[HS - 1]],
                              rows1_v, sem1).wait()
        sca(HS - 1, rows1_v)

    plsc.subcore_barrier()
    pltpu.sync_copy(acc_sh.at[pl.ds(s * ROWS_PER_TILE, ROWS_PER_TILE)],
                    out_hbm.at[c, pl.ds(s * ROWS_PER_TILE, ROWS_PER_TILE)])


_acc_kernel = functools.partial(
    pl.kernel,
    out_type=jax.ShapeDtypeStruct((NC, N_PAD, D), jnp.float32),
    mesh=_mesh,
    scratch_types=[
        pltpu.VMEM((HS, STEP), jnp.int32),
        pltpu.VMEM((HS, STEP), jnp.int32),
        pltpu.VMEM((STEP, D), jnp.float32),
        pltpu.VMEM((STEP, D), jnp.float32),
        pltpu.VMEM_SHARED((N_PAD, D), jnp.float32),
        pltpu.SemaphoreType.DMA,
        pltpu.SemaphoreType.DMA,
    ],
)(_acc_body)


_BLK = 1024
_GRID = N_PAD // _BLK


def _tcmm_body(x_ref, w_ref, h_ref):
    h_ref[...] = jnp.dot(x_ref[...], w_ref[...],
                         preferred_element_type=jnp.float32)


def _tcmm(x_pad, W1):
    return pl.pallas_call(
        _tcmm_body,
        grid=(_GRID,),
        in_specs=[
            pl.BlockSpec((_BLK, D), lambda i: (i, 0)),
            pl.BlockSpec((D, D), lambda i: (0, 0)),
        ],
        out_specs=pl.BlockSpec((_BLK, D), lambda i: (i, 0)),
        out_shape=jax.ShapeDtypeStruct((N_PAD, D), jnp.float32),
    )(x_pad, W1)


def _tc1_body(p_ref, h_ref, dinv_ref, h1s_ref):
    p = p_ref[...]
    deg = p[0, :, 0:1] + p[1, :, 0:1] + 1.0
    dinv = lax.rsqrt(deg)
    dinv_ref[...] = dinv
    h1s_ref[...] = dinv * h_ref[...]


def _tc1(deg_part, h1):
    return pl.pallas_call(
        _tc1_body,
        grid=(_GRID,),
        in_specs=[
            pl.BlockSpec((NC, _BLK, 16), lambda i: (0, i, 0)),
            pl.BlockSpec((_BLK, D), lambda i: (i, 0)),
        ],
        out_specs=[
            pl.BlockSpec((_BLK, 1), lambda i: (i, 0)),
            pl.BlockSpec((_BLK, D), lambda i: (i, 0)),
        ],
        out_shape=[
            jax.ShapeDtypeStruct((N_PAD, 1), jnp.float32),
            jax.ShapeDtypeStruct((N_PAD, D), jnp.float32),
        ],
    )(deg_part, h1)


def _tc2_body(p_ref, h1s_ref, dinv_ref, b_ref, w_ref, h2s_ref):
    acc = p_ref[0] + p_ref[1]
    dinv = dinv_ref[...]
    out1 = dinv * (acc + h1s_ref[...]) + b_ref[...]
    h2s_ref[...] = dinv * jnp.dot(out1, w_ref[...], preferred_element_type=jnp.float32)


def _tc2(p, h1s, dinv, b1, W2):
    return pl.pallas_call(
        _tc2_body,
        grid=(_GRID,),
        in_specs=[
            pl.BlockSpec((NC, _BLK, D), lambda i: (0, i, 0)),
            pl.BlockSpec((_BLK, D), lambda i: (i, 0)),
            pl.BlockSpec((_BLK, 1), lambda i: (i, 0)),
            pl.BlockSpec((D,), lambda i: (0,)),
            pl.BlockSpec((D, D), lambda i: (0, 0)),
        ],
        out_specs=pl.BlockSpec((_BLK, D), lambda i: (i, 0)),
        out_shape=jax.ShapeDtypeStruct((N_PAD, D), jnp.float32),
    )(p, h1s, dinv, b1, W2)


def _tc3_body(q_ref, h2s_ref, dinv_ref, b_ref, out_ref):
    acc = q_ref[0] + q_ref[1]
    out_ref[...] = dinv_ref[...] * (acc + h2s_ref[...]) + b_ref[...]


def _tc3(q, h2s, dinv, b2):
    return pl.pallas_call(
        _tc3_body,
        grid=(_GRID,),
        in_specs=[
            pl.BlockSpec((NC, _BLK, D), lambda i: (0, i, 0)),
            pl.BlockSpec((_BLK, D), lambda i: (i, 0)),
            pl.BlockSpec((_BLK, 1), lambda i: (i, 0)),
            pl.BlockSpec((D,), lambda i: (0,)),
        ],
        out_specs=pl.BlockSpec((_BLK, D), lambda i: (i, 0)),
        out_shape=jax.ShapeDtypeStruct((N_PAD, D), jnp.float32),
    )(q, h2s, dinv, b2)


def kernel(x, edge_index, W1, b1, W2, b2):
    src = edge_index[0].astype(jnp.int32)
    dst = edge_index[1].astype(jnp.int32)
    npad = NW * EPW_PAD - N_EDGES
    pad_ids = jnp.arange(npad, dtype=jnp.int32)
    src_p = jnp.concatenate([src, pad_ids % N_NODES]).reshape(NW, NH, HS, STEP)
    dst_p = jnp.concatenate([dst, N_NODES + pad_ids % (N_PAD - N_NODES)]
                            ).reshape(NW, NH, HS, STEP)
    x_pad = jnp.pad(x, ((0, N_PAD - N_NODES), (0, 0)))

    h1 = _tcmm(x_pad, W1)
    deg_part = _deg_kernel(dst_p.reshape(NW, NSTEP, STEP))
    dinv, h1s = _tc1(deg_part, h1)
    p1 = _acc_kernel(h1s, src_p, dst_p)
    h2s = _tc2(p1, h1s, dinv, b1, W2)
    p2 = _acc_kernel(h2s, src_p, dst_p)
    out = _tc3(p2, h2s, dinv, b2)
    return out[:N_NODES]
